# baseline (speedup 1.0000x reference)
import functools

import jax
import jax.numpy as jnp
from jax import lax
from jax.experimental import pallas as pl
from jax.experimental.pallas import tpu as pltpu
from jax.experimental.pallas import tpu_sc as plsc

USER = 5000
ITEM = 5000
N = USER + ITEM
NP = 10240
NNZ = 160000
H = 128
NTILES = 16
B = 128
NB = 80
EPT = NB * B
NNZP = NTILES * EPT
NBT = NTILES * NB
RPT = NP // NTILES
ZCH = 32

_mesh = plsc.VectorSubcoreMesh(core_axis_name="c", subcore_axis_name="s")


@functools.partial(
    pl.kernel,
    out_type=(
        jax.ShapeDtypeStruct((2 * NP, H), jnp.float32),
        jax.ShapeDtypeStruct((2 * NP, H), jnp.float32),
        jax.ShapeDtypeStruct((2 * NP, H), jnp.float32),
        jax.ShapeDtypeStruct((2 * NP, H), jnp.float32),
    ),
    mesh=_mesh,
    scratch_types=[
        pltpu.VMEM((2, B), jnp.int32),
        pltpu.VMEM((2, B), jnp.int32),
        pltpu.VMEM((2, B), jnp.int32),
        pltpu.VMEM((2, B), jnp.int32),
        pltpu.VMEM((1, B), jnp.float32),
        pltpu.VMEM((1, B), jnp.float32),
        pltpu.VMEM((1, B), jnp.float32),
        pltpu.VMEM((1, B), jnp.float32),
        pltpu.VMEM((B,), jnp.int32),
        pltpu.VMEM((B,), jnp.int32),
        pltpu.VMEM((B, H), jnp.float32),
        pltpu.VMEM((B, H), jnp.float32),
        pltpu.VMEM((ZCH, H), jnp.float32),
        pltpu.VMEM((ZCH, H), jnp.float32),
        pltpu.VMEM_SHARED((NP, H), jnp.float32),
        pltpu.SemaphoreType.DMA,
        pltpu.SemaphoreType.DMA,
        pltpu.SemaphoreType.DMA,
        pltpu.SemaphoreType.DMA,
        pltpu.SemaphoreType.DMA,
        pltpu.SemaphoreType.DMA,
        pltpu.SemaphoreType.DMA,
        pltpu.SemaphoreType.DMA,
        pltpu.SemaphoreType.DMA,
        pltpu.SemaphoreType.DMA,
        pltpu.SemaphoreType.DMA,
        pltpu.SemaphoreType.DMA,
    ],
)
def _propagate(e0, cvr, valr, e1, e2, e3, mout,
               cv0, cv1, cv2, cv3, vb0, vb1, vb2, vb3,
               ix0, ix1, gb0, gb1, abuf, bbuf, acc,
               cs0, cs1, cs2, cs3, vs0, vs1, vs2, vs3, gs0, gs1, ss0, ss1):
    c = lax.axis_index("c")
    s = lax.axis_index("s")
    sc_off = c * NP
    cvs = (cv0, cv1, cv2, cv3)
    css = (cs0, cs1, cs2, cs3)
    vbs = (vb0, vb1, vb2, vb3)
    vss = (vs0, vs1, vs2, vs3)
    ixs = (ix0, ix1)
    gbs = (gb0, gb1)
    gss = (gs0, gs1)
    sss = (ss0, ss1)

    def fire_cv(j, slot):
        pltpu.async_copy(cvr.at[s * NB + j], cvs[slot], css[slot])
        pltpu.async_copy(valr.at[s * NB + j], vbs[slot], vss[slot])

    def wait_cv(slot):
        pltpu.make_async_copy(cvr.at[s * NB], cvs[slot], css[slot]).wait()
        pltpu.make_async_copy(valr.at[s * NB], vbs[slot], vss[slot]).wait()

    zero16 = jnp.zeros((16,), jnp.float32)

    def _zfill(i, carry):
        for cc in range(H // 16):
            abuf[i, pl.ds(cc * 16, 16)] = zero16
        return carry
    lax.fori_loop(0, ZCH, _zfill, 0)

    for esrc, edst in ((e0, e1), (e1, e2), (e2, e3)):
        for k in range(RPT // ZCH):
            pltpu.sync_copy(abuf, acc.at[pl.ds(s * RPT + k * ZCH, ZCH)])
        plsc.subcore_barrier()

        fire_cv(0, 0)
        fire_cv(1, 1)
        pltpu.async_copy(esrc.at[pl.ds(0, B)], gb1, ss1)
        wait_cv(0)
        for cc in range(B // 16):
            sl = pl.ds(cc * 16, 16)
            ix0[sl] = cv0[0, sl] + sc_off
        pltpu.async_copy(esrc.at[ix0], gb0, gs0)

        def _quad(jj, carry):
            for bq in range(4):
                j = jj * 4 + bq
                ms = bq
                msn = (bq + 1) % 4
                msp = (bq + 2) % 4
                gsl = bq % 2
                gsn = (bq + 1) % 2
                cvj = cvs[ms]
                cvn = cvs[msn]

                wait_cv(msn)
                for cc in range(B // 16):
                    sl = pl.ds(cc * 16, 16)
                    ixs[gsn][sl] = cvn[0, sl] + sc_off
                pltpu.make_async_copy(
                    gbs[gsn], acc.at[cvn.at[1]], sss[gsn]).wait()
                pltpu.async_copy(esrc.at[ixs[gsn]], gbs[gsn], gss[gsn])
                fire_cv(j + 2, msp)

                gbj = gbs[gsl]
                pltpu.make_async_copy(esrc.at[pl.ds(0, B)], gbj,
                                      gss[gsl]).wait()

                vbj = vbs[ms]

                def _scale(g, icarry):
                    vals = vbj[0, pl.ds(g * 16, 16)]
                    for t in range(16):
                        vv = vals[t]
                        i = g * 16 + t
                        for cc in range(H // 16):
                            sl = pl.ds(cc * 16, 16)
                            gbj[i, sl] = gbj[i, sl] * vv
                    return icarry
                lax.fori_loop(0, B // 16, _scale, 0)

                pltpu.async_copy(gbj, acc.at[cvj.at[1]], sss[gsl], add=True)
            return carry
        lax.fori_loop(0, NB // 4, _quad, 0)

        pltpu.make_async_copy(esrc.at[pl.ds(0, B)], gb0, gs0).wait()
        wait_cv(1)
        pltpu.make_async_copy(gb1, acc.at[cv0.at[1]], ss1).wait()

        plsc.subcore_barrier()

        pltpu.sync_copy(acc.at[pl.ds(s * RPT, RPT)],
                        edst.at[pl.ds(sc_off + s * RPT, RPT)])

    for k in range(RPT // ZCH):
        base = sc_off + s * RPT + k * ZCH
        pltpu.sync_copy(e1.at[pl.ds(base, ZCH)], abuf)
        pltpu.sync_copy(e2.at[pl.ds(base, ZCH)], bbuf)
        pltpu.sync_copy(e3.at[pl.ds(base, ZCH)], gb0.at[pl.ds(0, ZCH)])

        def _avg(i, carry):
            for cc in range(H // 16):
                sl = pl.ds(cc * 16, 16)
                abuf[i, sl] = (abuf[i, sl] + bbuf[i, sl] + gb0[i, sl]) * (
                    1.0 / 3.0)
            return carry
        lax.fori_loop(0, ZCH, _avg, 0)
        pltpu.sync_copy(abuf, mout.at[pl.ds(base, ZCH)])


def kernel(user_emb, item_emb, adj_val, adj_row, adj_col, input1, input2):
    ego = jnp.concatenate([user_emb, item_emb], axis=0)
    zpad = jnp.zeros((NP - N, H), jnp.float32)
    e0 = jnp.concatenate([ego[:, :H], zpad, ego[:, H:], zpad], axis=0)

    pad = NNZP - NNZ
    colp = jnp.concatenate(
        [adj_col, jnp.zeros((pad,), jnp.int32)]).reshape(NBT, 1, B)
    rowp = jnp.concatenate(
        [adj_row, jnp.zeros((pad,), jnp.int32)]).reshape(NBT, 1, B)
    valp = jnp.concatenate(
        [adj_val, jnp.zeros((pad,), jnp.float32)]).reshape(NBT, 1, B)
    cvr = jnp.concatenate([colp, rowp], axis=1)
    cvr = jnp.concatenate(
        [cvr, jnp.zeros((2, 2, B), jnp.int32)], axis=0)
    valr = jnp.concatenate(
        [valp, jnp.zeros((2, 1, B), jnp.float32)], axis=0)

    _, _, _, m = _propagate(e0, cvr, valr)
    full = jnp.concatenate([m[:N], m[NP:NP + N]], axis=1)
    u = full[:USER]
    i = full[USER:]
    return (u, i, u, i)


# speedup vs baseline: 1.0141x; 1.0141x over previous
import functools

import jax
import jax.numpy as jnp
from jax import lax
from jax.experimental import pallas as pl
from jax.experimental.pallas import tpu as pltpu
from jax.experimental.pallas import tpu_sc as plsc

USER = 5000
ITEM = 5000
N = USER + ITEM
NP = 10240
NNZ = 160000
H = 128
NTILES = 16
B = 128
NB = 80
EPT = NB * B
NNZP = NTILES * EPT
NBT = NTILES * NB
RPT = NP // NTILES
ZCH = 32

_mesh = plsc.VectorSubcoreMesh(core_axis_name="c", subcore_axis_name="s")


@functools.partial(
    pl.kernel,
    out_type=(
        jax.ShapeDtypeStruct((2 * NP, H), jnp.float32),
        jax.ShapeDtypeStruct((2 * NP, H), jnp.float32),
        jax.ShapeDtypeStruct((2 * NP, H), jnp.float32),
    ),
    mesh=_mesh,
    scratch_types=[
        pltpu.VMEM((2, B), jnp.int32),
        pltpu.VMEM((2, B), jnp.int32),
        pltpu.VMEM((2, B), jnp.int32),
        pltpu.VMEM((2, B), jnp.int32),
        pltpu.VMEM((1, B), jnp.float32),
        pltpu.VMEM((1, B), jnp.float32),
        pltpu.VMEM((1, B), jnp.float32),
        pltpu.VMEM((1, B), jnp.float32),
        pltpu.VMEM((B,), jnp.int32),
        pltpu.VMEM((B,), jnp.int32),
        pltpu.VMEM((B, H), jnp.float32),
        pltpu.VMEM((B, H), jnp.float32),
        pltpu.VMEM((ZCH, H), jnp.float32),
        pltpu.VMEM((ZCH, H), jnp.float32),
        pltpu.VMEM_SHARED((NP, H), jnp.float32),
        pltpu.SemaphoreType.DMA,
        pltpu.SemaphoreType.DMA,
        pltpu.SemaphoreType.DMA,
        pltpu.SemaphoreType.DMA,
        pltpu.SemaphoreType.DMA,
        pltpu.SemaphoreType.DMA,
        pltpu.SemaphoreType.DMA,
        pltpu.SemaphoreType.DMA,
        pltpu.SemaphoreType.DMA,
        pltpu.SemaphoreType.DMA,
        pltpu.SemaphoreType.DMA,
        pltpu.SemaphoreType.DMA,
    ],
)
def _propagate(e0, cvr, valr, zr, e1, e2, mout,
               cv0, cv1, cv2, cv3, vb0, vb1, vb2, vb3,
               ix0, ix1, gb0, gb1, abuf, bbuf, acc,
               cs0, cs1, cs2, cs3, vs0, vs1, vs2, vs3, gs0, gs1, ss0, ss1):
    c = lax.axis_index("c")
    s = lax.axis_index("s")
    sc_off = c * NP
    cvs = (cv0, cv1, cv2, cv3)
    css = (cs0, cs1, cs2, cs3)
    vbs = (vb0, vb1, vb2, vb3)
    vss = (vs0, vs1, vs2, vs3)
    ixs = (ix0, ix1)
    gbs = (gb0, gb1)
    gss = (gs0, gs1)
    sss = (ss0, ss1)

    def fire_cv(j, slot):
        pltpu.async_copy(cvr.at[s * NB + j], cvs[slot], css[slot])
        pltpu.async_copy(valr.at[s * NB + j], vbs[slot], vss[slot])

    def wait_cv(slot):
        pltpu.make_async_copy(cvr.at[s * NB], cvs[slot], css[slot]).wait()
        pltpu.make_async_copy(valr.at[s * NB], vbs[slot], vss[slot]).wait()

    for esrc, edst in ((e0, e1), (e1, e2), (e2, None)):
        pltpu.sync_copy(zr, acc.at[pl.ds(s * RPT, RPT)])
        plsc.subcore_barrier()

        fire_cv(0, 0)
        fire_cv(1, 1)
        pltpu.async_copy(esrc.at[pl.ds(0, B)], gb1, ss1)
        wait_cv(0)
        for cc in range(B // 16):
            sl = pl.ds(cc * 16, 16)
            ix0[sl] = cv0[0, sl] + sc_off
        pltpu.async_copy(esrc.at[ix0], gb0, gs0)

        def _quad(jj, carry):
            for bq in range(4):
                j = jj * 4 + bq
                ms = bq
                msn = (bq + 1) % 4
                msp = (bq + 2) % 4
                gsl = bq % 2
                gsn = (bq + 1) % 2
                cvj = cvs[ms]
                cvn = cvs[msn]

                wait_cv(msn)
                for cc in range(B // 16):
                    sl = pl.ds(cc * 16, 16)
                    ixs[gsn][sl] = cvn[0, sl] + sc_off
                pltpu.make_async_copy(
                    gbs[gsn], acc.at[cvn.at[1]], sss[gsn]).wait()
                pltpu.async_copy(esrc.at[ixs[gsn]], gbs[gsn], gss[gsn])
                fire_cv(j + 2, msp)

                gbj = gbs[gsl]
                pltpu.make_async_copy(esrc.at[pl.ds(0, B)], gbj,
                                      gss[gsl]).wait()

                vbj = vbs[ms]

                def _scale(g, icarry):
                    vals = vbj[0, pl.ds(g * 16, 16)]
                    for t in range(16):
                        vv = vals[t]
                        i = g * 16 + t
                        for cc in range(H // 16):
                            sl = pl.ds(cc * 16, 16)
                            gbj[i, sl] = gbj[i, sl] * vv
                    return icarry
                lax.fori_loop(0, B // 16, _scale, 0)

                pltpu.async_copy(gbj, acc.at[cvj.at[1]], sss[gsl], add=True)
            return carry
        lax.fori_loop(0, NB // 4, _quad, 0)

        pltpu.make_async_copy(esrc.at[pl.ds(0, B)], gb0, gs0).wait()
        wait_cv(1)
        pltpu.make_async_copy(gb1, acc.at[cv0.at[1]], ss1).wait()

        plsc.subcore_barrier()

        if edst is not None:
            pltpu.sync_copy(acc.at[pl.ds(s * RPT, RPT)],
                            edst.at[pl.ds(sc_off + s * RPT, RPT)])

    for k in range(RPT // ZCH):
        loc = s * RPT + k * ZCH
        base = sc_off + loc
        pltpu.sync_copy(e1.at[pl.ds(base, ZCH)], abuf)
        pltpu.sync_copy(e2.at[pl.ds(base, ZCH)], bbuf)
        pltpu.sync_copy(acc.at[pl.ds(loc, ZCH)], gb0.at[pl.ds(0, ZCH)])

        def _avg(i, carry):
            for cc in range(H // 16):
                sl = pl.ds(cc * 16, 16)
                abuf[i, sl] = (abuf[i, sl] + bbuf[i, sl] + gb0[i, sl]) * (
                    1.0 / 3.0)
            return carry
        lax.fori_loop(0, ZCH, _avg, 0)
        pltpu.sync_copy(abuf, mout.at[pl.ds(base, ZCH)])


def kernel(user_emb, item_emb, adj_val, adj_row, adj_col, input1, input2):
    ego = jnp.concatenate([user_emb, item_emb], axis=0)
    zpad = jnp.zeros((NP - N, H), jnp.float32)
    e0 = jnp.concatenate([ego[:, :H], zpad, ego[:, H:], zpad], axis=0)

    pad = NNZP - NNZ
    colp = jnp.concatenate(
        [adj_col, jnp.zeros((pad,), jnp.int32)]).reshape(NBT, 1, B)
    rowp = jnp.concatenate(
        [adj_row, jnp.zeros((pad,), jnp.int32)]).reshape(NBT, 1, B)
    valp = jnp.concatenate(
        [adj_val, jnp.zeros((pad,), jnp.float32)]).reshape(NBT, 1, B)
    cvr = jnp.concatenate([colp, rowp], axis=1)
    cvr = jnp.concatenate(
        [cvr, jnp.zeros((2, 2, B), jnp.int32)], axis=0)
    valr = jnp.concatenate(
        [valp, jnp.zeros((2, 1, B), jnp.float32)], axis=0)

    zr = jnp.zeros((RPT, H), jnp.float32)
    _, _, m = _propagate(e0, cvr, valr, zr)
    full = jnp.concatenate([m[:N], m[NP:NP + N]], axis=1)
    u = full[:USER]
    i = full[USER:]
    return (u, i, u, i)
